# Initial kernel scaffold; baseline (speedup 1.0000x reference)
import jax
import jax.numpy as jnp
from jax.experimental import pallas as pl


def kernel(inputs, pos_table):
    raise NotImplementedError("write your pallas kernel here")


# speedup vs baseline: 1.2758x; 1.2758x over previous
import jax
import jax.numpy as jnp
from jax.experimental import pallas as pl


def _add_kernel(x_ref, p_ref, o_ref):
    o_ref[...] = x_ref[...] + p_ref[...]


def kernel(inputs, pos_table):
    B, S, D = inputs.shape
    BS = 512
    x = inputs.reshape(B * S, D)
    nblk = (B * S) // BS
    pblk = S // BS
    out = pl.pallas_call(
        _add_kernel,
        grid=(nblk,),
        in_specs=[
            pl.BlockSpec((BS, D), lambda i: (i, 0)),
            pl.BlockSpec((BS, D), lambda i: (i % pblk, 0)),
        ],
        out_specs=pl.BlockSpec((BS, D), lambda i: (i, 0)),
        out_shape=jax.ShapeDtypeStruct((B * S, D), inputs.dtype),
    )(x, pos_table)
    return out.reshape(B, S, D)


# speedup vs baseline: 1.4873x; 1.1657x over previous
import jax
import jax.numpy as jnp
from jax.experimental import pallas as pl


def _add_kernel(x_ref, p_ref, o_ref):
    o_ref[...] = x_ref[...] + p_ref[...]


def kernel(inputs, pos_table):
    B, S, D = inputs.shape
    BS = 512
    nblk = S // BS
    out = pl.pallas_call(
        _add_kernel,
        grid=(nblk, B),
        in_specs=[
            pl.BlockSpec((1, BS, D), lambda i, j: (j, i, 0)),
            pl.BlockSpec((BS, D), lambda i, j: (i, 0)),
        ],
        out_specs=pl.BlockSpec((1, BS, D), lambda i, j: (j, i, 0)),
        out_shape=jax.ShapeDtypeStruct((B, S, D), inputs.dtype),
    )(inputs, pos_table)
    return out


# speedup vs baseline: 1.6614x; 1.1171x over previous
import jax
import jax.numpy as jnp
from jax.experimental import pallas as pl


def _add_kernel(x_ref, p_ref, o_ref):
    o_ref[...] = x_ref[...] + p_ref[...]


def kernel(inputs, pos_table):
    B, S, D = inputs.shape
    BS = 1024
    nblk = S // BS
    out = pl.pallas_call(
        _add_kernel,
        grid=(nblk, B),
        in_specs=[
            pl.BlockSpec((1, BS, D), lambda i, j: (j, i, 0)),
            pl.BlockSpec((BS, D), lambda i, j: (i, 0)),
        ],
        out_specs=pl.BlockSpec((1, BS, D), lambda i, j: (j, i, 0)),
        out_shape=jax.ShapeDtypeStruct((B, S, D), inputs.dtype),
    )(inputs, pos_table)
    return out


# speedup vs baseline: 1.7339x; 1.0437x over previous
import jax
import jax.numpy as jnp
from jax.experimental import pallas as pl


def _add_kernel(x_ref, p_ref, o_ref):
    o_ref[...] = x_ref[...] + p_ref[...]


def kernel(inputs, pos_table):
    B, S, D = inputs.shape
    BS = 2048
    nblk = S // BS
    out = pl.pallas_call(
        _add_kernel,
        grid=(nblk, B),
        in_specs=[
            pl.BlockSpec((1, BS, D), lambda i, j: (j, i, 0)),
            pl.BlockSpec((BS, D), lambda i, j: (i, 0)),
        ],
        out_specs=pl.BlockSpec((1, BS, D), lambda i, j: (j, i, 0)),
        out_shape=jax.ShapeDtypeStruct((B, S, D), inputs.dtype),
    )(inputs, pos_table)
    return out
